# baseline (speedup 1.0000x reference)
import functools

import jax
import jax.numpy as jnp
from jax import lax
from jax.experimental import pallas as pl
from jax.experimental.pallas import tpu as pltpu
from jax.experimental.pallas import tpu_sc as plsc

BATCH = 16384
VOCAB = 100000
EMBED = 64
NW = 32
BPW = BATCH // NW

_TC = 16384
_TGRID = -(-VOCAB // _TC)

_mesh = plsc.VectorSubcoreMesh(core_axis_name="c", subcore_axis_name="s")


def _pack2_body(a_ref, b_ref, out_ref):
    out_ref[...] = jnp.concatenate([a_ref[...].T, b_ref[...].T], axis=1)


_HB = 51200
_HC = 12800


def _pack_halves(tTa):
    return pl.pallas_call(
        _pack2_body,
        grid=(_HB // _HC,),
        in_specs=[
            pl.BlockSpec((EMBED, _HC), lambda j: (0, j)),
            pl.BlockSpec((EMBED, _HC), lambda j: (0, j + _HB // _HC)),
        ],
        out_specs=pl.BlockSpec((_HC, 2 * EMBED), lambda j: (j, 0)),
        out_shape=jax.ShapeDtypeStruct((_HB, 2 * EMBED), jnp.float32),
    )(tTa, tTa)


def _pack2(tTa, tTb):
    return pl.pallas_call(
        _pack2_body,
        grid=(_TGRID,),
        in_specs=[
            pl.BlockSpec((EMBED, _TC), lambda j: (0, j)),
            pl.BlockSpec((EMBED, _TC), lambda j: (0, j)),
        ],
        out_specs=pl.BlockSpec((_TC, 2 * EMBED), lambda j: (j, 0)),
        out_shape=jax.ShapeDtypeStruct((VOCAB, 2 * EMBED), jnp.float32),
    )(tTa, tTb)


@functools.partial(
    pl.kernel,
    mesh=_mesh,
    out_type=jax.ShapeDtypeStruct((BATCH, 2 * EMBED), jnp.float32),
    scratch_types=[
        pltpu.VMEM((BPW,), jnp.int32),
        pltpu.VMEM((BPW, 2 * EMBED), jnp.float32),
        pltpu.SemaphoreType.DMA,
    ],
)
def _gather_one(idx_hbm, table_hbm, out_hbm, idx_v, rows_v, sem):
    wid = lax.axis_index("s") * 2 + lax.axis_index("c")
    base = wid * BPW
    pltpu.sync_copy(idx_hbm.at[pl.ds(base, BPW)], idx_v)
    pltpu.async_copy(table_hbm.at[idx_v], rows_v, sem).wait()
    pltpu.sync_copy(rows_v, out_hbm.at[pl.ds(base, BPW)])


@functools.partial(
    pl.kernel,
    mesh=_mesh,
    out_type=(
        jax.ShapeDtypeStruct((BATCH, 2 * EMBED), jnp.float32),
        jax.ShapeDtypeStruct((BATCH, 2 * EMBED), jnp.float32),
    ),
    scratch_types=[
        pltpu.VMEM((BPW,), jnp.int32),
        pltpu.VMEM((BPW,), jnp.int32),
        pltpu.VMEM((BPW, 2 * EMBED), jnp.float32),
        pltpu.VMEM((BPW, 2 * EMBED), jnp.float32),
        pltpu.SemaphoreType.DMA,
        pltpu.SemaphoreType.DMA,
    ],
)
def _gather_two(idx_hbm, table_hbm, oa_hbm, ob_hbm,
                ia_v, ib_v, ra_v, rb_v, sa, sb):
    wid = lax.axis_index("s") * 2 + lax.axis_index("c")
    base = wid * BPW
    pltpu.sync_copy(idx_hbm.at[pl.ds(base, BPW)], ia_v)
    pltpu.sync_copy(idx_hbm.at[pl.ds(BATCH + base, BPW)], ib_v)
    ca = pltpu.async_copy(table_hbm.at[ia_v], ra_v, sa)
    cb = pltpu.async_copy(table_hbm.at[ib_v], rb_v, sb)
    ca.wait()
    pltpu.sync_copy(ra_v, oa_hbm.at[pl.ds(base, BPW)])
    cb.wait()
    pltpu.sync_copy(rb_v, ob_hbm.at[pl.ds(base, BPW)])


def kernel(x, emb_mi, emb_mo, emb_mtext):
    xT = jnp.transpose(x)
    x2 = xT[2]
    hi2 = x2 >= _HB
    idx2 = jnp.where(hi2, x2 - _HB, x2)
    rows01 = _pack2(jnp.transpose(emb_mi), jnp.transpose(emb_mo))
    o0 = _gather_one(xT[0], rows01)
    o1 = _gather_one(xT[1], rows01)
    rows2h = _pack_halves(jnp.transpose(emb_mtext))
    o2 = _gather_one(idx2, rows2h)
    p2 = jnp.where(hi2[:, None], o2[:, EMBED:], o2[:, :EMBED])
    return jnp.concatenate([o0[:, :EMBED], o1[:, EMBED:], p2], axis=1)


# speedup vs baseline: 1.0264x; 1.0264x over previous
import functools

import jax
import jax.numpy as jnp
from jax import lax
from jax.experimental import pallas as pl
from jax.experimental.pallas import tpu as pltpu
from jax.experimental.pallas import tpu_sc as plsc

BATCH = 16384
VOCAB = 100000
EMBED = 64
NW = 32
BPW = BATCH // NW

_TC = 16384
_TGRID = -(-VOCAB // _TC)

_mesh = plsc.VectorSubcoreMesh(core_axis_name="c", subcore_axis_name="s")


def _pack2_body(a_ref, b_ref, out_ref):
    out_ref[...] = jnp.concatenate([a_ref[...].T, b_ref[...].T], axis=1)


def _pack2(tTa, tTb):
    return pl.pallas_call(
        _pack2_body,
        grid=(_TGRID,),
        in_specs=[
            pl.BlockSpec((EMBED, _TC), lambda j: (0, j)),
            pl.BlockSpec((EMBED, _TC), lambda j: (0, j)),
        ],
        out_specs=pl.BlockSpec((_TC, 2 * EMBED), lambda j: (j, 0)),
        out_shape=jax.ShapeDtypeStruct((VOCAB, 2 * EMBED), jnp.float32),
    )(tTa, tTb)


@functools.partial(
    pl.kernel,
    mesh=_mesh,
    out_type=jax.ShapeDtypeStruct((BATCH, 2 * EMBED), jnp.float32),
    scratch_types=[
        pltpu.VMEM((BPW,), jnp.int32),
        pltpu.VMEM((BPW, 2 * EMBED), jnp.float32),
        pltpu.SemaphoreType.DMA,
    ],
)
def _gather_one(idx_hbm, table_hbm, out_hbm, idx_v, rows_v, sem):
    wid = lax.axis_index("s") * 2 + lax.axis_index("c")
    base = wid * BPW
    pltpu.sync_copy(idx_hbm.at[pl.ds(base, BPW)], idx_v)
    pltpu.async_copy(table_hbm.at[idx_v], rows_v, sem).wait()
    pltpu.sync_copy(rows_v, out_hbm.at[pl.ds(base, BPW)])


def _pack_dup_body(a_ref, out_ref):
    at = a_ref[...].T
    out_ref[...] = jnp.concatenate([at, at], axis=1)


def _pack_dup(tTa):
    return pl.pallas_call(
        _pack_dup_body,
        grid=(_TGRID,),
        in_specs=[pl.BlockSpec((EMBED, _TC), lambda j: (0, j))],
        out_specs=pl.BlockSpec((_TC, 2 * EMBED), lambda j: (j, 0)),
        out_shape=jax.ShapeDtypeStruct((VOCAB, 2 * EMBED), jnp.float32),
    )(tTa)


def kernel(x, emb_mi, emb_mo, emb_mtext):
    xT = jnp.transpose(x)
    rows01 = _pack2(jnp.transpose(emb_mi), jnp.transpose(emb_mo))
    o0 = _gather_one(xT[0], rows01)
    o1 = _gather_one(xT[1], rows01)
    rows22 = _pack_dup(jnp.transpose(emb_mtext))
    o2 = _gather_one(xT[2], rows22)
    return jnp.concatenate(
        [o0[:, :EMBED], o1[:, EMBED:], o2[:, :EMBED]], axis=1)


# speedup vs baseline: 1.0639x; 1.0366x over previous
import functools

import jax
import jax.numpy as jnp
from jax import lax
from jax.experimental import pallas as pl
from jax.experimental.pallas import tpu as pltpu
from jax.experimental.pallas import tpu_sc as plsc

BATCH = 16384
VOCAB = 100000
EMBED = 64
NW = 32
BPW = BATCH // NW

_TC = 20096
_TGRID = -(-VOCAB // _TC)

_mesh = plsc.VectorSubcoreMesh(core_axis_name="c", subcore_axis_name="s")


def _pack2_body(a_ref, b_ref, out_ref):
    out_ref[...] = jnp.concatenate([a_ref[...].T, b_ref[...].T], axis=1)


def _pack2(tTa, tTb):
    return pl.pallas_call(
        _pack2_body,
        grid=(_TGRID,),
        in_specs=[
            pl.BlockSpec((EMBED, _TC), lambda j: (0, j)),
            pl.BlockSpec((EMBED, _TC), lambda j: (0, j)),
        ],
        out_specs=pl.BlockSpec((_TC, 2 * EMBED), lambda j: (j, 0)),
        out_shape=jax.ShapeDtypeStruct((VOCAB, 2 * EMBED), jnp.float32),
    )(tTa, tTb)


@functools.partial(
    pl.kernel,
    mesh=_mesh,
    out_type=jax.ShapeDtypeStruct((BATCH, 2 * EMBED), jnp.float32),
    scratch_types=[
        pltpu.VMEM((BPW,), jnp.int32),
        pltpu.VMEM((BPW, 2 * EMBED), jnp.float32),
        pltpu.SemaphoreType.DMA,
    ],
)
def _gather_one(idx_hbm, table_hbm, out_hbm, idx_v, rows_v, sem):
    wid = lax.axis_index("s") * 2 + lax.axis_index("c")
    base = wid * BPW
    pltpu.sync_copy(idx_hbm.at[pl.ds(base, BPW)], idx_v)
    pltpu.async_copy(table_hbm.at[idx_v], rows_v, sem).wait()
    pltpu.sync_copy(rows_v, out_hbm.at[pl.ds(base, BPW)])


def _pack_dup_body(a_ref, out_ref):
    at = a_ref[...].T
    out_ref[...] = jnp.concatenate([at, at], axis=1)


def _pack_dup(tTa):
    return pl.pallas_call(
        _pack_dup_body,
        grid=(_TGRID,),
        in_specs=[pl.BlockSpec((EMBED, _TC), lambda j: (0, j))],
        out_specs=pl.BlockSpec((_TC, 2 * EMBED), lambda j: (j, 0)),
        out_shape=jax.ShapeDtypeStruct((VOCAB, 2 * EMBED), jnp.float32),
    )(tTa)


def kernel(x, emb_mi, emb_mo, emb_mtext):
    xT = jnp.transpose(x)
    rows01 = _pack2(jnp.transpose(emb_mi), jnp.transpose(emb_mo))
    o0 = _gather_one(xT[0], rows01)
    o1 = _gather_one(xT[1], rows01)
    rows22 = _pack_dup(jnp.transpose(emb_mtext))
    o2 = _gather_one(xT[2], rows22)
    return jnp.concatenate(
        [o0[:, :EMBED], o1[:, EMBED:], o2[:, :EMBED]], axis=1)
